# baseline (speedup 1.0000x reference)
import functools

import jax
import jax.numpy as jnp
import numpy as np
from jax import lax
from jax.experimental import pallas as pl
from jax.experimental.pallas import tpu as pltpu
from jax.experimental.pallas import tpu_sc as plsc

D_MODEL = 768
D_FF = 3072
E = 8
N = 2048
TILE = 256
NT = 24
NPAD = NT * TILE
NTP = 32
NW = 32
TPW = N // NW
CHUNK = 32
LANES = 768 // 16


def _router_body(x_ref, gw_ref, lng_ref, lnb_ref,
                 xn_ref, d1_ref, d2_ref, w1e_ref, w2e_ref, te_ref, ta_ref, aux_ref):
    f32 = jnp.float32
    x = x_ref[...]
    mu = jnp.mean(x, axis=1, keepdims=True)
    xc = x - mu
    var = jnp.mean(xc * xc, axis=1, keepdims=True)
    xn = xc * lax.rsqrt(var + 1e-5) * lng_ref[...] + lnb_ref[...]
    xn_ref[...] = xn
    logits = lax.dot_general(xn, gw_ref[...], (((1,), (1,)), ((), ())),
                             preferred_element_type=f32)
    m = jnp.max(logits, axis=1, keepdims=True)
    ex = jnp.exp(logits - m)
    probs = ex / jnp.sum(ex, axis=1, keepdims=True)
    iota = lax.broadcasted_iota(jnp.int32, (N, E), 1)
    p1 = jnp.max(probs, axis=1, keepdims=True)
    i1 = jnp.min(jnp.where(probs == p1, iota, E), axis=1, keepdims=True)
    probs2 = jnp.where(iota == i1, -1.0, probs)
    p2 = jnp.max(probs2, axis=1, keepdims=True)
    i2 = jnp.min(jnp.where(probs2 == p2, iota, E), axis=1, keepdims=True)
    s = p1 + p2
    w1e_ref[...] = jnp.broadcast_to(p1 / s, (N, 16))
    w2e_ref[...] = jnp.broadcast_to(p2 / s, (N, 16))
    importance = jnp.sum(probs, axis=0, keepdims=True)
    oh1 = (iota == i1).astype(f32)
    oh2 = (iota == i2).astype(f32)
    onehots = oh1 + oh2
    load = jnp.sum(onehots, axis=0, keepdims=True)
    aux = jnp.sum(importance * load) * (E / (N * N + 1e-6))
    aux_ref[...] = jnp.reshape(aux, (1, 1))
    cum = onehots
    sft = 1
    while sft < N:
        cum = cum + jnp.concatenate(
            [jnp.zeros((sft, E), f32), cum[:N - sft]], axis=0)
        sft *= 2
    exc = cum - onehots
    rank1 = jnp.sum(exc * oh1, axis=1, keepdims=True)
    rank2 = jnp.sum(exc * oh2, axis=1, keepdims=True)
    padded = jnp.ceil(load / TILE) * TILE
    off = padded
    sft = 1
    while sft < E:
        off = off + jnp.concatenate(
            [jnp.zeros((1, sft), f32), off[:, :E - sft]], axis=1)
        sft *= 2
    offs_exc = off - padded
    total = off[0, E - 1]
    d1 = jnp.sum(offs_exc * oh1, axis=1, keepdims=True) + rank1
    d2 = jnp.sum(offs_exc * oh2, axis=1, keepdims=True) + rank2
    d1_ref[...] = d1.astype(jnp.int32)
    d2_ref[...] = d2.astype(jnp.int32)
    tstart = lax.broadcasted_iota(jnp.int32, (NTP, 1), 0).astype(f32) * TILE
    cnt = jnp.sum((offs_exc <= tstart).astype(f32), axis=1, keepdims=True)
    te_ref[...] = (cnt - 1.0).astype(jnp.int32)
    ta_ref[...] = (tstart < total).astype(jnp.int32)


def _router(x, gate_w, ln_g, ln_b):
    f32, i32 = jnp.float32, jnp.int32
    outs = [
        jax.ShapeDtypeStruct((N, D_MODEL), f32),
        jax.ShapeDtypeStruct((N, 1), i32),
        jax.ShapeDtypeStruct((N, 1), i32),
        jax.ShapeDtypeStruct((N, 16), f32),
        jax.ShapeDtypeStruct((N, 16), f32),
        jax.ShapeDtypeStruct((NTP, 1), i32),
        jax.ShapeDtypeStruct((NTP, 1), i32),
        jax.ShapeDtypeStruct((1, 1), f32),
    ]
    return pl.pallas_call(_router_body, out_shape=outs)(
        x, gate_w, ln_g.reshape(1, D_MODEL), ln_b.reshape(1, D_MODEL))


def _sc_scatter(xn, d1, d2):
    mesh = plsc.VectorSubcoreMesh(core_axis_name="c", subcore_axis_name="s")

    @functools.partial(
        pl.kernel, mesh=mesh,
        out_type=jax.ShapeDtypeStruct((NPAD, D_MODEL), jnp.float32),
        scratch_types=[
            pltpu.VMEM((TPW,), jnp.int32),
            pltpu.VMEM((TPW,), jnp.int32),
            pltpu.VMEM((TPW, D_MODEL), jnp.float32),
            pltpu.SemaphoreType.DMA,
            pltpu.SemaphoreType.DMA,
        ],
    )
    def body(xn_hbm, d1_hbm, d2_hbm, xs_hbm, idx1, idx2, rows, sem1, sem2):
        wid = lax.axis_index("s") * 2 + lax.axis_index("c")
        base = wid * TPW
        pltpu.sync_copy(xn_hbm.at[pl.ds(base, TPW)], rows)
        pltpu.sync_copy(d1_hbm.at[pl.ds(base, TPW)], idx1)
        pltpu.sync_copy(d2_hbm.at[pl.ds(base, TPW)], idx2)
        c1 = pltpu.async_copy(rows, xs_hbm.at[idx1], sem1)
        c2 = pltpu.async_copy(rows, xs_hbm.at[idx2], sem2)
        c1.wait()
        c2.wait()

    return body(xn, d1, d2)


def _gelu(h):
    return 0.5 * h * (1.0 + lax.erf(h * np.float32(1.0 / np.sqrt(2.0))))


NSPLIT = 4
FSPLIT = D_FF // NSPLIT


def _ffn_body(te_ref, ta_ref, xs_ref, *rest):
    w1_refs = rest[:NSPLIT]
    b1_ref = rest[NSPLIT]
    w2_refs = rest[NSPLIT + 1:2 * NSPLIT + 1]
    b2_ref = rest[2 * NSPLIT + 1]
    ys_ref = rest[2 * NSPLIT + 2]
    act = ta_ref[pl.program_id(0)]

    @pl.when(act == 1)
    def _():
        xs = xs_ref[...]
        y = b2_ref[0]
        for j in range(NSPLIT):
            h = lax.dot_general(xs, w1_refs[j][0], (((1,), (1,)), ((), ())),
                                preferred_element_type=jnp.float32)
            h = _gelu(h + b1_ref[0, :, pl.ds(j * FSPLIT, FSPLIT)])
            y = y + lax.dot_general(h, w2_refs[j][0], (((1,), (1,)), ((), ())),
                                    preferred_element_type=jnp.float32)
        ys_ref[...] = y


def _ffn(te, ta, xs, W1, b1, W2, b2):
    w1_specs = [
        pl.BlockSpec((1, FSPLIT, D_MODEL),
                     functools.partial(lambda j, i, te, ta: (te[i], j, 0), j))
        for j in range(NSPLIT)
    ]
    w2_specs = [
        pl.BlockSpec((1, D_MODEL, FSPLIT),
                     functools.partial(lambda j, i, te, ta: (te[i], 0, j), j))
        for j in range(NSPLIT)
    ]
    grid_spec = pltpu.PrefetchScalarGridSpec(
        num_scalar_prefetch=2,
        grid=(NT,),
        in_specs=[
            pl.BlockSpec((TILE, D_MODEL), lambda i, te, ta: (i, 0)),
            *w1_specs,
            pl.BlockSpec((1, 1, D_FF), lambda i, te, ta: (te[i], 0, 0)),
            *w2_specs,
            pl.BlockSpec((1, 1, D_MODEL), lambda i, te, ta: (te[i], 0, 0)),
        ],
        out_specs=pl.BlockSpec((TILE, D_MODEL), lambda i, te, ta: (i, 0)),
    )
    return pl.pallas_call(
        _ffn_body, grid_spec=grid_spec,
        out_shape=jax.ShapeDtypeStruct((NPAD, D_MODEL), jnp.float32),
    )(te, ta, xs, *([W1] * NSPLIT), b1.reshape(E, 1, D_FF),
      *([W2] * NSPLIT), b2.reshape(E, 1, D_MODEL))


def _sc_combine(x, ys, d1, d2, w1e, w2e):
    mesh = plsc.VectorSubcoreMesh(core_axis_name="c", subcore_axis_name="s")

    @functools.partial(
        pl.kernel, mesh=mesh,
        out_type=jax.ShapeDtypeStruct((N, D_MODEL), jnp.float32),
        scratch_types=[
            pltpu.VMEM((CHUNK,), jnp.int32),
            pltpu.VMEM((CHUNK,), jnp.int32),
            pltpu.VMEM((CHUNK, 16), jnp.float32),
            pltpu.VMEM((CHUNK, 16), jnp.float32),
            pltpu.VMEM((CHUNK, D_MODEL), jnp.float32),
            pltpu.VMEM((CHUNK, D_MODEL), jnp.float32),
            pltpu.VMEM((CHUNK, D_MODEL), jnp.float32),
            pltpu.SemaphoreType.DMA,
            pltpu.SemaphoreType.DMA,
        ],
    )
    def body(x_hbm, ys_hbm, d1_hbm, d2_hbm, w1_hbm, w2_hbm, out_hbm,
             idx1, idx2, w1r, w2r, xrows, buf1, buf2, sem1, sem2):
        wid = lax.axis_index("s") * 2 + lax.axis_index("c")
        for cnk in range(TPW // CHUNK):
            base = wid * TPW + cnk * CHUNK
            pltpu.sync_copy(x_hbm.at[pl.ds(base, CHUNK)], xrows)
            pltpu.sync_copy(d1_hbm.at[pl.ds(base, CHUNK)], idx1)
            pltpu.sync_copy(d2_hbm.at[pl.ds(base, CHUNK)], idx2)
            pltpu.sync_copy(w1_hbm.at[pl.ds(base, CHUNK)], w1r)
            pltpu.sync_copy(w2_hbm.at[pl.ds(base, CHUNK)], w2r)
            c1 = pltpu.async_copy(ys_hbm.at[idx1], buf1, sem1)
            c2 = pltpu.async_copy(ys_hbm.at[idx2], buf2, sem2)
            c1.wait()
            c2.wait()

            def tok(t, carry):
                ws1 = w1r[t, :]
                ws2 = w2r[t, :]
                for c in range(LANES):
                    sl = pl.ds(c * 16, 16)
                    xrows[t, sl] = (xrows[t, sl] + ws1 * buf1[t, sl]
                                    + ws2 * buf2[t, sl])
                return carry

            lax.fori_loop(0, CHUNK, tok, 0)
            pltpu.sync_copy(xrows, out_hbm.at[pl.ds(base, CHUNK)])

    return body(x, ys, d1, d2, w1e, w2e)


def kernel(x, gate_w, ln_g, ln_b, W1, b1, W2, b2):
    xn, d1, d2, w1e, w2e, te, ta, aux = _router(x, gate_w, ln_g, ln_b)
    d1f = d1[:, 0]
    d2f = d2[:, 0]
    xs = _sc_scatter(xn, d1f, d2f)
    ys = _ffn(te[:, 0], ta[:, 0], xs, W1, b1, W2, b2)
    out = _sc_combine(x, ys, d1f, d2f, w1e, w2e)
    return out, aux[0, 0]


# speedup vs baseline: 1.1384x; 1.1384x over previous
import functools

import jax
import jax.numpy as jnp
import numpy as np
from jax import lax
from jax.experimental import pallas as pl
from jax.experimental.pallas import tpu as pltpu
from jax.experimental.pallas import tpu_sc as plsc

D_MODEL = 768
D_FF = 3072
E = 8
N = 2048
TILE = 256
NT = 24
NPAD = NT * TILE
NTP = 32
NW = 32
TPW = N // NW
CHUNK = 32
LANES = 768 // 16


def _router_body(x_ref, gw_ref, lng_ref, lnb_ref,
                 xn_ref, d1_ref, d2_ref, w1e_ref, w2e_ref, te_ref, ta_ref,
                 xo_ref, sl_ref, np_ref, aux_ref):
    f32 = jnp.float32
    x = x_ref[...]
    mu = jnp.mean(x, axis=1, keepdims=True)
    xc = x - mu
    var = jnp.mean(xc * xc, axis=1, keepdims=True)
    xn = xc * lax.rsqrt(var + 1e-5) * lng_ref[...] + lnb_ref[...]
    xn_ref[...] = xn
    logits = lax.dot_general(xn, gw_ref[...], (((1,), (1,)), ((), ())),
                             preferred_element_type=f32)
    m = jnp.max(logits, axis=1, keepdims=True)
    ex = jnp.exp(logits - m)
    probs = ex / jnp.sum(ex, axis=1, keepdims=True)
    iota = lax.broadcasted_iota(jnp.int32, (N, E), 1)
    p1 = jnp.max(probs, axis=1, keepdims=True)
    i1 = jnp.min(jnp.where(probs == p1, iota, E), axis=1, keepdims=True)
    probs2 = jnp.where(iota == i1, -1.0, probs)
    p2 = jnp.max(probs2, axis=1, keepdims=True)
    i2 = jnp.min(jnp.where(probs2 == p2, iota, E), axis=1, keepdims=True)
    s = p1 + p2
    w1e_ref[...] = jnp.broadcast_to(p1 / s, (N, 16))
    w2e_ref[...] = jnp.broadcast_to(p2 / s, (N, 16))
    importance = jnp.sum(probs, axis=0, keepdims=True)
    oh1 = (iota == i1).astype(f32)
    oh2 = (iota == i2).astype(f32)
    onehots = oh1 + oh2
    load = jnp.sum(onehots, axis=0, keepdims=True)
    aux = jnp.sum(importance * load) * (E / (N * N + 1e-6))
    aux_ref[...] = jnp.reshape(aux, (1, 1))
    cum = onehots
    sft = 1
    while sft < N:
        cum = cum + jnp.concatenate(
            [jnp.zeros((sft, E), f32), cum[:N - sft]], axis=0)
        sft *= 2
    exc = cum - onehots
    rank1 = jnp.sum(exc * oh1, axis=1, keepdims=True)
    rank2 = jnp.sum(exc * oh2, axis=1, keepdims=True)
    padded = jnp.ceil(load / TILE) * TILE
    off = padded
    sft = 1
    while sft < E:
        off = off + jnp.concatenate(
            [jnp.zeros((1, sft), f32), off[:, :E - sft]], axis=1)
        sft *= 2
    offs_exc = off - padded
    total = off[0, E - 1]
    d1 = jnp.sum(offs_exc * oh1, axis=1, keepdims=True) + rank1
    d2 = jnp.sum(offs_exc * oh2, axis=1, keepdims=True) + rank2
    d1_ref[...] = d1.astype(jnp.int32)
    d2_ref[...] = d2.astype(jnp.int32)
    tstart = lax.broadcasted_iota(jnp.int32, (NTP, 1), 0).astype(f32) * TILE
    cnt = jnp.sum((offs_exc <= tstart).astype(f32), axis=1, keepdims=True)
    te_ref[...] = (cnt - 1.0).astype(jnp.int32)
    ta_ref[...] = (tstart < total).astype(jnp.int32)
    ia8 = lax.broadcasted_iota(jnp.int32, (E, E), 0)
    ib8 = lax.broadcasted_iota(jnp.int32, (E, E), 1)
    eye8 = (ia8 == ib8).astype(f32)
    xo_ref[...] = lax.dot_general(
        eye8, offs_exc * (1.0 / TILE), (((1,), (1,)), ((), ())),
        preferred_element_type=f32).astype(jnp.int32)
    prs = (padded > 0).astype(f32)
    rk = prs
    sft = 1
    while sft < E:
        rk = rk + jnp.concatenate(
            [jnp.zeros((1, sft), f32), rk[:, :E - sft]], axis=1)
        sft *= 2
    rank = rk - 1.0
    slot_lane = rank - 2.0 * jnp.floor(rank * 0.5)
    sl_ref[...] = lax.dot_general(
        eye8, slot_lane, (((1,), (1,)), ((), ())),
        preferred_element_type=f32).astype(jnp.int32)
    prs_b = jnp.broadcast_to(prs, (E, E))
    mm = jnp.where((ib8 > ia8) & (prs_b > 0.5), ib8, 99)
    np_ref[...] = jnp.min(mm, axis=1, keepdims=True)


def _router(x, gate_w, ln_g, ln_b):
    f32, i32 = jnp.float32, jnp.int32
    outs = [
        jax.ShapeDtypeStruct((N, D_MODEL), f32),
        jax.ShapeDtypeStruct((N, 1), i32),
        jax.ShapeDtypeStruct((N, 1), i32),
        jax.ShapeDtypeStruct((N, 16), f32),
        jax.ShapeDtypeStruct((N, 16), f32),
        jax.ShapeDtypeStruct((NTP, 1), i32),
        jax.ShapeDtypeStruct((NTP, 1), i32),
        jax.ShapeDtypeStruct((E, 1), i32),
        jax.ShapeDtypeStruct((E, 1), i32),
        jax.ShapeDtypeStruct((E, 1), i32),
        jax.ShapeDtypeStruct((1, 1), f32),
    ]
    return pl.pallas_call(_router_body, out_shape=outs)(
        x, gate_w, ln_g.reshape(1, D_MODEL), ln_b.reshape(1, D_MODEL))


def _sc_scatter(xn, d1, d2):
    mesh = plsc.VectorSubcoreMesh(core_axis_name="c", subcore_axis_name="s")

    @functools.partial(
        pl.kernel, mesh=mesh,
        out_type=jax.ShapeDtypeStruct((NPAD, D_MODEL), jnp.float32),
        scratch_types=[
            pltpu.VMEM((TPW,), jnp.int32),
            pltpu.VMEM((TPW,), jnp.int32),
            pltpu.VMEM((TPW, D_MODEL), jnp.float32),
            pltpu.SemaphoreType.DMA,
            pltpu.SemaphoreType.DMA,
        ],
    )
    def body(xn_hbm, d1_hbm, d2_hbm, xs_hbm, idx1, idx2, rows, sem1, sem2):
        wid = lax.axis_index("s") * 2 + lax.axis_index("c")
        base = wid * TPW
        pltpu.sync_copy(xn_hbm.at[pl.ds(base, TPW)], rows)
        pltpu.sync_copy(d1_hbm.at[pl.ds(base, TPW)], idx1)
        pltpu.sync_copy(d2_hbm.at[pl.ds(base, TPW)], idx2)
        c1 = pltpu.async_copy(rows, xs_hbm.at[idx1], sem1)
        c2 = pltpu.async_copy(rows, xs_hbm.at[idx2], sem2)
        c1.wait()
        c2.wait()

    return body(xn, d1, d2)


def _gelu(h):
    return 0.5 * h * (1.0 + lax.erf(h * np.float32(1.0 / np.sqrt(2.0))))


def _ffn_body(te_ref, ta_ref, xo_ref, sl_ref, np_ref,
              xs_ref, w1_hbm, b1_ref, w2_hbm, b2_ref, ys_ref,
              w1buf, w2buf, sem1, sem2):
    i = pl.program_id(0)
    act = ta_ref[i]

    @pl.when(act == 1)
    def _():
        e = te_ref[i]
        s = sl_ref[e]

        @pl.when(i == xo_ref[e])
        def _():
            @pl.when(i == 0)
            def _():
                pltpu.make_async_copy(w1_hbm.at[e], w1buf.at[0], sem1.at[0]).start()
                pltpu.make_async_copy(w2_hbm.at[e], w2buf.at[0], sem2.at[0]).start()

            nxt = np_ref[e]

            @pl.when(nxt < E)
            def _():
                pltpu.make_async_copy(w1_hbm.at[nxt], w1buf.at[1 - s],
                                      sem1.at[1 - s]).start()
                pltpu.make_async_copy(w2_hbm.at[nxt], w2buf.at[1 - s],
                                      sem2.at[1 - s]).start()

            pltpu.make_async_copy(w1_hbm.at[e], w1buf.at[s], sem1.at[s]).wait()
            pltpu.make_async_copy(w2_hbm.at[e], w2buf.at[s], sem2.at[s]).wait()

        xs = xs_ref[...]
        h = lax.dot_general(xs, w1buf[s], (((1,), (1,)), ((), ())),
                            preferred_element_type=jnp.float32)
        h = _gelu(h + b1_ref[0])
        y = lax.dot_general(h, w2buf[s], (((1,), (1,)), ((), ())),
                            preferred_element_type=jnp.float32)
        ys_ref[...] = y + b2_ref[0]


def _ffn(te, ta, xo, sl, npx, xs, W1, b1, W2, b2):
    grid_spec = pltpu.PrefetchScalarGridSpec(
        num_scalar_prefetch=5,
        grid=(NT,),
        in_specs=[
            pl.BlockSpec((TILE, D_MODEL), lambda i, *_: (i, 0)),
            pl.BlockSpec(memory_space=pl.ANY),
            pl.BlockSpec((1, 1, D_FF), lambda i, te, *_: (te[i], 0, 0)),
            pl.BlockSpec(memory_space=pl.ANY),
            pl.BlockSpec((1, 1, D_MODEL), lambda i, te, *_: (te[i], 0, 0)),
        ],
        out_specs=pl.BlockSpec((TILE, D_MODEL), lambda i, *_: (i, 0)),
        scratch_shapes=[
            pltpu.VMEM((2, D_FF, D_MODEL), jnp.float32),
            pltpu.VMEM((2, D_MODEL, D_FF), jnp.float32),
            pltpu.SemaphoreType.DMA((2,)),
            pltpu.SemaphoreType.DMA((2,)),
        ],
    )
    return pl.pallas_call(
        _ffn_body, grid_spec=grid_spec,
        out_shape=jax.ShapeDtypeStruct((NPAD, D_MODEL), jnp.float32),
    )(te, ta, xo, sl, npx, xs, W1, b1.reshape(E, 1, D_FF),
      W2, b2.reshape(E, 1, D_MODEL))


def _sc_combine(x, ys, d1, d2, w1e, w2e):
    mesh = plsc.VectorSubcoreMesh(core_axis_name="c", subcore_axis_name="s")

    @functools.partial(
        pl.kernel, mesh=mesh,
        out_type=jax.ShapeDtypeStruct((N, D_MODEL), jnp.float32),
        scratch_types=[
            pltpu.VMEM((CHUNK,), jnp.int32),
            pltpu.VMEM((CHUNK,), jnp.int32),
            pltpu.VMEM((CHUNK, 16), jnp.float32),
            pltpu.VMEM((CHUNK, 16), jnp.float32),
            pltpu.VMEM((CHUNK, D_MODEL), jnp.float32),
            pltpu.VMEM((CHUNK, D_MODEL), jnp.float32),
            pltpu.VMEM((CHUNK, D_MODEL), jnp.float32),
            pltpu.SemaphoreType.DMA,
            pltpu.SemaphoreType.DMA,
        ],
    )
    def body(x_hbm, ys_hbm, d1_hbm, d2_hbm, w1_hbm, w2_hbm, out_hbm,
             idx1, idx2, w1r, w2r, xrows, buf1, buf2, sem1, sem2):
        wid = lax.axis_index("s") * 2 + lax.axis_index("c")
        for cnk in range(TPW // CHUNK):
            base = wid * TPW + cnk * CHUNK
            pltpu.sync_copy(x_hbm.at[pl.ds(base, CHUNK)], xrows)
            pltpu.sync_copy(d1_hbm.at[pl.ds(base, CHUNK)], idx1)
            pltpu.sync_copy(d2_hbm.at[pl.ds(base, CHUNK)], idx2)
            pltpu.sync_copy(w1_hbm.at[pl.ds(base, CHUNK)], w1r)
            pltpu.sync_copy(w2_hbm.at[pl.ds(base, CHUNK)], w2r)
            c1 = pltpu.async_copy(ys_hbm.at[idx1], buf1, sem1)
            c2 = pltpu.async_copy(ys_hbm.at[idx2], buf2, sem2)
            c1.wait()
            c2.wait()

            def tok(t, carry):
                ws1 = w1r[t, :]
                ws2 = w2r[t, :]
                for c in range(LANES):
                    sl = pl.ds(c * 16, 16)
                    xrows[t, sl] = (xrows[t, sl] + ws1 * buf1[t, sl]
                                    + ws2 * buf2[t, sl])
                return carry

            lax.fori_loop(0, CHUNK, tok, 0)
            pltpu.sync_copy(xrows, out_hbm.at[pl.ds(base, CHUNK)])

    return body(x, ys, d1, d2, w1e, w2e)


def kernel(x, gate_w, ln_g, ln_b, W1, b1, W2, b2):
    xn, d1, d2, w1e, w2e, te, ta, xo, sl, npx, aux = _router(x, gate_w, ln_g, ln_b)
    d1f = d1[:, 0]
    d2f = d2[:, 0]
    xs = _sc_scatter(xn, d1f, d2f)
    ys = _ffn(te[:, 0], ta[:, 0], xo[:, 0], sl[:, 0], npx[:, 0],
              xs, W1, b1, W2, b2)
    out = _sc_combine(x, ys, d1f, d2f, w1e, w2e)
    return out, aux[0, 0]


# speedup vs baseline: 1.1574x; 1.0168x over previous
import functools

import jax
import jax.numpy as jnp
import numpy as np
from jax import lax
from jax.experimental import pallas as pl
from jax.experimental.pallas import tpu as pltpu
from jax.experimental.pallas import tpu_sc as plsc

D_MODEL = 768
D_FF = 3072
E = 8
N = 2048
TILE = 256
NT = 24
NPAD = NT * TILE
NTP = 32
NW = 32
TPW = N // NW
CHUNK = 32
LANES = 768 // 16


def _router_body(x_ref, gw_ref, lng_ref, lnb_ref,
                 xn_ref, d1_ref, d2_ref, w1e_ref, w2e_ref, te_ref, ta_ref,
                 xo_ref, sl_ref, np_ref, aux_ref):
    f32 = jnp.float32
    x = x_ref[...]
    mu = jnp.mean(x, axis=1, keepdims=True)
    xc = x - mu
    var = jnp.mean(xc * xc, axis=1, keepdims=True)
    xn = xc * lax.rsqrt(var + 1e-5) * lng_ref[...] + lnb_ref[...]
    xn_ref[...] = xn
    logits = lax.dot_general(xn, gw_ref[...], (((1,), (1,)), ((), ())),
                             preferred_element_type=f32)
    m = jnp.max(logits, axis=1, keepdims=True)
    ex = jnp.exp(logits - m)
    probs = ex / jnp.sum(ex, axis=1, keepdims=True)
    iota = lax.broadcasted_iota(jnp.int32, (N, E), 1)
    p1 = jnp.max(probs, axis=1, keepdims=True)
    i1 = jnp.min(jnp.where(probs == p1, iota, E), axis=1, keepdims=True)
    probs2 = jnp.where(iota == i1, -1.0, probs)
    p2 = jnp.max(probs2, axis=1, keepdims=True)
    i2 = jnp.min(jnp.where(probs2 == p2, iota, E), axis=1, keepdims=True)
    s = p1 + p2
    w1e_ref[...] = jnp.broadcast_to(p1 / s, (N, 16))
    w2e_ref[...] = jnp.broadcast_to(p2 / s, (N, 16))
    importance = jnp.sum(probs, axis=0, keepdims=True)
    oh1 = (iota == i1).astype(f32)
    oh2 = (iota == i2).astype(f32)
    onehots = oh1 + oh2
    load = jnp.sum(onehots, axis=0, keepdims=True)
    aux = jnp.sum(importance * load) * (E / (N * N + 1e-6))
    aux_ref[...] = jnp.reshape(aux, (1, 1))
    cum = onehots
    sft = 1
    while sft < N:
        cum = cum + jnp.concatenate(
            [jnp.zeros((sft, E), f32), cum[:N - sft]], axis=0)
        sft *= 2
    exc = cum - onehots
    rank1 = jnp.sum(exc * oh1, axis=1, keepdims=True)
    rank2 = jnp.sum(exc * oh2, axis=1, keepdims=True)
    padded = jnp.ceil(load / TILE) * TILE
    off = padded
    sft = 1
    while sft < E:
        off = off + jnp.concatenate(
            [jnp.zeros((1, sft), f32), off[:, :E - sft]], axis=1)
        sft *= 2
    offs_exc = off - padded
    total = off[0, E - 1]
    d1 = jnp.sum(offs_exc * oh1, axis=1, keepdims=True) + rank1
    d2 = jnp.sum(offs_exc * oh2, axis=1, keepdims=True) + rank2
    d1_ref[...] = d1.astype(jnp.int32)
    d2_ref[...] = d2.astype(jnp.int32)
    tstart = lax.broadcasted_iota(jnp.int32, (NTP, 1), 0).astype(f32) * TILE
    cnt = jnp.sum((offs_exc <= tstart).astype(f32), axis=1, keepdims=True)
    te_ref[...] = (cnt - 1.0).astype(jnp.int32)
    ta_ref[...] = (tstart < total).astype(jnp.int32)
    ia8 = lax.broadcasted_iota(jnp.int32, (E, E), 0)
    ib8 = lax.broadcasted_iota(jnp.int32, (E, E), 1)
    eye8 = (ia8 == ib8).astype(f32)
    xo_ref[...] = lax.dot_general(
        eye8, offs_exc * (1.0 / TILE), (((1,), (1,)), ((), ())),
        preferred_element_type=f32).astype(jnp.int32)
    prs = (padded > 0).astype(f32)
    rk = prs
    sft = 1
    while sft < E:
        rk = rk + jnp.concatenate(
            [jnp.zeros((1, sft), f32), rk[:, :E - sft]], axis=1)
        sft *= 2
    rank = rk - 1.0
    slot_lane = rank - 2.0 * jnp.floor(rank * 0.5)
    sl_ref[...] = lax.dot_general(
        eye8, slot_lane, (((1,), (1,)), ((), ())),
        preferred_element_type=f32).astype(jnp.int32)
    prs_b = jnp.broadcast_to(prs, (E, E))
    mm = jnp.where((ib8 > ia8) & (prs_b > 0.5), ib8, 99)
    np_ref[...] = jnp.min(mm, axis=1, keepdims=True)


def _router(x, gate_w, ln_g, ln_b):
    f32, i32 = jnp.float32, jnp.int32
    outs = [
        jax.ShapeDtypeStruct((N, D_MODEL), f32),
        jax.ShapeDtypeStruct((N, 1), i32),
        jax.ShapeDtypeStruct((N, 1), i32),
        jax.ShapeDtypeStruct((N, 16), f32),
        jax.ShapeDtypeStruct((N, 16), f32),
        jax.ShapeDtypeStruct((NTP, 1), i32),
        jax.ShapeDtypeStruct((NTP, 1), i32),
        jax.ShapeDtypeStruct((E, 1), i32),
        jax.ShapeDtypeStruct((E, 1), i32),
        jax.ShapeDtypeStruct((E, 1), i32),
        jax.ShapeDtypeStruct((1, 1), f32),
    ]
    return pl.pallas_call(_router_body, out_shape=outs)(
        x, gate_w, ln_g.reshape(1, D_MODEL), ln_b.reshape(1, D_MODEL))


def _sc_scatter(xn, d1, d2):
    mesh = plsc.VectorSubcoreMesh(core_axis_name="c", subcore_axis_name="s")

    @functools.partial(
        pl.kernel, mesh=mesh,
        out_type=jax.ShapeDtypeStruct((NPAD, D_MODEL), jnp.float32),
        scratch_types=[
            pltpu.VMEM((TPW,), jnp.int32),
            pltpu.VMEM((TPW,), jnp.int32),
            pltpu.VMEM((TPW, D_MODEL), jnp.float32),
            pltpu.SemaphoreType.DMA,
            pltpu.SemaphoreType.DMA,
        ],
    )
    def body(xn_hbm, d1_hbm, d2_hbm, xs_hbm, idx1, idx2, rows, sem1, sem2):
        wid = lax.axis_index("s") * 2 + lax.axis_index("c")
        base = wid * TPW
        pltpu.sync_copy(xn_hbm.at[pl.ds(base, TPW)], rows)
        pltpu.sync_copy(d1_hbm.at[pl.ds(base, TPW)], idx1)
        pltpu.sync_copy(d2_hbm.at[pl.ds(base, TPW)], idx2)
        c1 = pltpu.async_copy(rows, xs_hbm.at[idx1], sem1)
        c2 = pltpu.async_copy(rows, xs_hbm.at[idx2], sem2)
        c1.wait()
        c2.wait()

    return body(xn, d1, d2)


def _gelu(h):
    return 0.5 * h * (1.0 + lax.erf(h * np.float32(1.0 / np.sqrt(2.0))))


def _ffn_body(te_ref, ta_ref, xo_ref, sl_ref, np_ref,
              xs_ref, w1_hbm, b1_ref, w2_hbm, b2_ref, ys_ref,
              w1buf, w2buf, sem1, sem2):
    i = pl.program_id(0)
    act = ta_ref[i]

    @pl.when(act == 1)
    def _():
        e = te_ref[i]
        s = sl_ref[e]

        @pl.when(i == xo_ref[e])
        def _():
            @pl.when(i == 0)
            def _():
                pltpu.make_async_copy(w1_hbm.at[e], w1buf.at[0], sem1.at[0]).start()
                pltpu.make_async_copy(w2_hbm.at[e], w2buf.at[0], sem2.at[0]).start()

            nxt = np_ref[e]

            @pl.when(nxt < E)
            def _():
                pltpu.make_async_copy(w1_hbm.at[nxt], w1buf.at[1 - s],
                                      sem1.at[1 - s]).start()
                pltpu.make_async_copy(w2_hbm.at[nxt], w2buf.at[1 - s],
                                      sem2.at[1 - s]).start()

            pltpu.make_async_copy(w1_hbm.at[e], w1buf.at[s], sem1.at[s]).wait()
            pltpu.make_async_copy(w2_hbm.at[e], w2buf.at[s], sem2.at[s]).wait()

        xs = xs_ref[...]
        h = lax.dot_general(xs, w1buf[s], (((1,), (1,)), ((), ())),
                            preferred_element_type=jnp.float32)
        h = _gelu(h + b1_ref[0])
        y = lax.dot_general(h, w2buf[s], (((1,), (1,)), ((), ())),
                            preferred_element_type=jnp.float32)
        ys_ref[...] = y + b2_ref[0]


def _ffn(te, ta, xo, sl, npx, xs, W1, b1, W2, b2):
    grid_spec = pltpu.PrefetchScalarGridSpec(
        num_scalar_prefetch=5,
        grid=(NT,),
        in_specs=[
            pl.BlockSpec((TILE, D_MODEL), lambda i, *_: (i, 0)),
            pl.BlockSpec(memory_space=pl.ANY),
            pl.BlockSpec((1, 1, D_FF), lambda i, te, *_: (te[i], 0, 0)),
            pl.BlockSpec(memory_space=pl.ANY),
            pl.BlockSpec((1, 1, D_MODEL), lambda i, te, *_: (te[i], 0, 0)),
        ],
        out_specs=pl.BlockSpec((TILE, D_MODEL), lambda i, *_: (i, 0)),
        scratch_shapes=[
            pltpu.VMEM((2, D_FF, D_MODEL), jnp.float32),
            pltpu.VMEM((2, D_MODEL, D_FF), jnp.float32),
            pltpu.SemaphoreType.DMA((2,)),
            pltpu.SemaphoreType.DMA((2,)),
        ],
    )
    return pl.pallas_call(
        _ffn_body, grid_spec=grid_spec,
        out_shape=jax.ShapeDtypeStruct((NPAD, D_MODEL), jnp.float32),
    )(te, ta, xo, sl, npx, xs, W1, b1.reshape(E, 1, D_FF),
      W2, b2.reshape(E, 1, D_MODEL))


CH = 16
NCH = TPW // CH


def _sc_combine(x, ys, d1, d2, w1e, w2e):
    mesh = plsc.VectorSubcoreMesh(core_axis_name="c", subcore_axis_name="s")

    per_slot = [
        pltpu.VMEM((CH,), jnp.int32),
        pltpu.VMEM((CH,), jnp.int32),
        pltpu.VMEM((CH, 16), jnp.float32),
        pltpu.VMEM((CH, 16), jnp.float32),
        pltpu.VMEM((CH, D_MODEL), jnp.float32),
        pltpu.VMEM((CH, D_MODEL), jnp.float32),
        pltpu.VMEM((CH, D_MODEL), jnp.float32),
        pltpu.SemaphoreType.DMA,
        pltpu.SemaphoreType.DMA,
    ]

    @functools.partial(
        pl.kernel, mesh=mesh,
        out_type=jax.ShapeDtypeStruct((N, D_MODEL), jnp.float32),
        scratch_types=per_slot + per_slot,
    )
    def body(x_hbm, ys_hbm, d1_hbm, d2_hbm, w1_hbm, w2_hbm, out_hbm, *scr):
        slots = [scr[:9], scr[9:]]
        wid = lax.axis_index("s") * 2 + lax.axis_index("c")

        def issue(cnk, slot):
            idx1, idx2, w1r, w2r, xrows, buf1, buf2, semx, semg = slots[slot]
            base = wid * TPW + cnk * CH
            hx = pltpu.async_copy(x_hbm.at[pl.ds(base, CH)], xrows, semx)
            pltpu.sync_copy(d1_hbm.at[pl.ds(base, CH)], idx1)
            pltpu.sync_copy(d2_hbm.at[pl.ds(base, CH)], idx2)
            pltpu.sync_copy(w1_hbm.at[pl.ds(base, CH)], w1r)
            pltpu.sync_copy(w2_hbm.at[pl.ds(base, CH)], w2r)
            h1 = pltpu.async_copy(ys_hbm.at[idx1], buf1, semg)
            h2 = pltpu.async_copy(ys_hbm.at[idx2], buf2, semg)
            return (hx, h1, h2)

        hs = issue(0, 0)
        for cnk in range(NCH):
            slot = cnk % 2
            nhs = issue(cnk + 1, 1 - slot) if cnk + 1 < NCH else None
            for h in hs:
                h.wait()
            _, _, w1r, w2r, xrows, buf1, buf2, _, _ = slots[slot]

            def tok(t, carry):
                ws1 = w1r[t, :]
                ws2 = w2r[t, :]
                for c in range(LANES):
                    sl = pl.ds(c * 16, 16)
                    xrows[t, sl] = (xrows[t, sl] + ws1 * buf1[t, sl]
                                    + ws2 * buf2[t, sl])
                return carry

            lax.fori_loop(0, CH, tok, 0)
            base = wid * TPW + cnk * CH
            pltpu.sync_copy(xrows, out_hbm.at[pl.ds(base, CH)])
            hs = nhs

    return body(x, ys, d1, d2, w1e, w2e)


def kernel(x, gate_w, ln_g, ln_b, W1, b1, W2, b2):
    xn, d1, d2, w1e, w2e, te, ta, xo, sl, npx, aux = _router(x, gate_w, ln_g, ln_b)
    d1f = d1[:, 0]
    d2f = d2[:, 0]
    xs = _sc_scatter(xn, d1f, d2f)
    ys = _ffn(te[:, 0], ta[:, 0], xo[:, 0], sl[:, 0], npx[:, 0],
              xs, W1, b1, W2, b2)
    out = _sc_combine(x, ys, d1f, d2f, w1e, w2e)
    return out, aux[0, 0]


# speedup vs baseline: 1.1697x; 1.0106x over previous
import functools

import jax
import jax.numpy as jnp
import numpy as np
from jax import lax
from jax.experimental import pallas as pl
from jax.experimental.pallas import tpu as pltpu
from jax.experimental.pallas import tpu_sc as plsc

D_MODEL = 768
D_FF = 3072
E = 8
N = 2048
TILE = 256
NT = 24
NPAD = NT * TILE
NTP = 32
NW = 32
TPW = N // NW
CHUNK = 32
LANES = 768 // 16


def _router_body(x_ref, gw_ref, lng_ref, lnb_ref,
                 xn_ref, d1_ref, d2_ref, w1e_ref, w2e_ref, te_ref, ta_ref,
                 xo_ref, sl_ref, np_ref, aux_ref):
    f32 = jnp.float32
    x = x_ref[...]
    mu = jnp.mean(x, axis=1, keepdims=True)
    xc = x - mu
    var = jnp.mean(xc * xc, axis=1, keepdims=True)
    xn = xc * lax.rsqrt(var + 1e-5) * lng_ref[...] + lnb_ref[...]
    xn_ref[...] = xn
    logits = lax.dot_general(xn, gw_ref[...], (((1,), (1,)), ((), ())),
                             preferred_element_type=f32)
    m = jnp.max(logits, axis=1, keepdims=True)
    ex = jnp.exp(logits - m)
    probs = ex / jnp.sum(ex, axis=1, keepdims=True)
    iota = lax.broadcasted_iota(jnp.int32, (N, E), 1)
    p1 = jnp.max(probs, axis=1, keepdims=True)
    i1 = jnp.min(jnp.where(probs == p1, iota, E), axis=1, keepdims=True)
    probs2 = jnp.where(iota == i1, -1.0, probs)
    p2 = jnp.max(probs2, axis=1, keepdims=True)
    i2 = jnp.min(jnp.where(probs2 == p2, iota, E), axis=1, keepdims=True)
    s = p1 + p2
    w1e_ref[...] = jnp.broadcast_to(p1 / s, (N, 16))
    w2e_ref[...] = jnp.broadcast_to(p2 / s, (N, 16))
    importance = jnp.sum(probs, axis=0, keepdims=True)
    oh1 = (iota == i1).astype(f32)
    oh2 = (iota == i2).astype(f32)
    onehots = oh1 + oh2
    load = jnp.sum(onehots, axis=0, keepdims=True)
    aux = jnp.sum(importance * load) * (E / (N * N + 1e-6))
    aux_ref[...] = jnp.reshape(aux, (1, 1))
    cum = onehots
    sft = 1
    while sft < N:
        cum = cum + jnp.concatenate(
            [jnp.zeros((sft, E), f32), cum[:N - sft]], axis=0)
        sft *= 2
    exc = cum - onehots
    rank1 = jnp.sum(exc * oh1, axis=1, keepdims=True)
    rank2 = jnp.sum(exc * oh2, axis=1, keepdims=True)
    padded = jnp.ceil(load / TILE) * TILE
    off = padded
    sft = 1
    while sft < E:
        off = off + jnp.concatenate(
            [jnp.zeros((1, sft), f32), off[:, :E - sft]], axis=1)
        sft *= 2
    offs_exc = off - padded
    total = off[0, E - 1]
    d1 = jnp.sum(offs_exc * oh1, axis=1, keepdims=True) + rank1
    d2 = jnp.sum(offs_exc * oh2, axis=1, keepdims=True) + rank2
    d1_ref[...] = d1.astype(jnp.int32)
    d2_ref[...] = d2.astype(jnp.int32)
    tstart = lax.broadcasted_iota(jnp.int32, (NTP, 1), 0).astype(f32) * TILE
    cnt = jnp.sum((offs_exc <= tstart).astype(f32), axis=1, keepdims=True)
    te_ref[...] = (cnt - 1.0).astype(jnp.int32)
    ta_ref[...] = (tstart < total).astype(jnp.int32)
    ia8 = lax.broadcasted_iota(jnp.int32, (E, E), 0)
    ib8 = lax.broadcasted_iota(jnp.int32, (E, E), 1)
    eye8 = (ia8 == ib8).astype(f32)
    xo_ref[...] = lax.dot_general(
        eye8, offs_exc * (1.0 / TILE), (((1,), (1,)), ((), ())),
        preferred_element_type=f32).astype(jnp.int32)
    prs = (padded > 0).astype(f32)
    rk = prs
    sft = 1
    while sft < E:
        rk = rk + jnp.concatenate(
            [jnp.zeros((1, sft), f32), rk[:, :E - sft]], axis=1)
        sft *= 2
    rank = rk - 1.0
    slot_lane = rank - 2.0 * jnp.floor(rank * 0.5)
    sl_ref[...] = lax.dot_general(
        eye8, slot_lane, (((1,), (1,)), ((), ())),
        preferred_element_type=f32).astype(jnp.int32)
    prs_b = jnp.broadcast_to(prs, (E, E))
    mm = jnp.where((ib8 > ia8) & (prs_b > 0.5), ib8, 99)
    np_ref[...] = jnp.min(mm, axis=1, keepdims=True)


def _router(x, gate_w, ln_g, ln_b):
    f32, i32 = jnp.float32, jnp.int32
    outs = [
        jax.ShapeDtypeStruct((N, D_MODEL), f32),
        jax.ShapeDtypeStruct((N, 1), i32),
        jax.ShapeDtypeStruct((N, 1), i32),
        jax.ShapeDtypeStruct((N, 16), f32),
        jax.ShapeDtypeStruct((N, 16), f32),
        jax.ShapeDtypeStruct((NTP, 1), i32),
        jax.ShapeDtypeStruct((NTP, 1), i32),
        jax.ShapeDtypeStruct((E, 1), i32),
        jax.ShapeDtypeStruct((E, 1), i32),
        jax.ShapeDtypeStruct((E, 1), i32),
        jax.ShapeDtypeStruct((1, 1), f32),
    ]
    return pl.pallas_call(_router_body, out_shape=outs)(
        x, gate_w, ln_g.reshape(1, D_MODEL), ln_b.reshape(1, D_MODEL))


def _sc_scatter(xn, d1, d2):
    mesh = plsc.VectorSubcoreMesh(core_axis_name="c", subcore_axis_name="s")

    @functools.partial(
        pl.kernel, mesh=mesh,
        out_type=jax.ShapeDtypeStruct((NPAD, D_MODEL), jnp.float32),
        scratch_types=[
            pltpu.VMEM((TPW,), jnp.int32),
            pltpu.VMEM((TPW,), jnp.int32),
            pltpu.VMEM((TPW, D_MODEL), jnp.float32),
            pltpu.SemaphoreType.DMA,
            pltpu.SemaphoreType.DMA,
        ],
    )
    def body(xn_hbm, d1_hbm, d2_hbm, xs_hbm, idx1, idx2, rows, sem1, sem2):
        wid = lax.axis_index("s") * 2 + lax.axis_index("c")
        base = wid * TPW
        pltpu.sync_copy(xn_hbm.at[pl.ds(base, TPW)], rows)
        pltpu.sync_copy(d1_hbm.at[pl.ds(base, TPW)], idx1)
        pltpu.sync_copy(d2_hbm.at[pl.ds(base, TPW)], idx2)
        c1 = pltpu.async_copy(rows, xs_hbm.at[idx1], sem1)
        c2 = pltpu.async_copy(rows, xs_hbm.at[idx2], sem2)
        c1.wait()
        c2.wait()

    return body(xn, d1, d2)


def _gelu(h):
    return 0.5 * h * (1.0 + lax.erf(h * np.float32(1.0 / np.sqrt(2.0))))


def _ffn_body(te_ref, ta_ref, xo_ref, sl_ref, np_ref,
              xs_ref, w1_hbm, b1_ref, w2_hbm, b2_ref, ys_ref,
              w1buf, w2buf, sem1, sem2):
    i = pl.program_id(0)
    act = ta_ref[i]

    @pl.when(act == 1)
    def _():
        e = te_ref[i]
        s = sl_ref[e]

        @pl.when(i == xo_ref[e])
        def _():
            @pl.when(i == 0)
            def _():
                pltpu.make_async_copy(w1_hbm.at[e], w1buf.at[0], sem1.at[0]).start()
                pltpu.make_async_copy(w2_hbm.at[e], w2buf.at[0], sem2.at[0]).start()

            nxt = np_ref[e]

            @pl.when(nxt < E)
            def _():
                pltpu.make_async_copy(w1_hbm.at[nxt], w1buf.at[1 - s],
                                      sem1.at[1 - s]).start()
                pltpu.make_async_copy(w2_hbm.at[nxt], w2buf.at[1 - s],
                                      sem2.at[1 - s]).start()

            pltpu.make_async_copy(w1_hbm.at[e], w1buf.at[s], sem1.at[s]).wait()
            pltpu.make_async_copy(w2_hbm.at[e], w2buf.at[s], sem2.at[s]).wait()

        xs = xs_ref[...]
        h = lax.dot_general(xs, w1buf[s], (((1,), (1,)), ((), ())),
                            preferred_element_type=jnp.float32)
        h = _gelu(h + b1_ref[0])
        y = lax.dot_general(h, w2buf[s], (((1,), (1,)), ((), ())),
                            preferred_element_type=jnp.float32)
        ys_ref[...] = y + b2_ref[0]


def _ffn(te, ta, xo, sl, npx, xs, W1, b1, W2, b2):
    grid_spec = pltpu.PrefetchScalarGridSpec(
        num_scalar_prefetch=5,
        grid=(NT,),
        in_specs=[
            pl.BlockSpec((TILE, D_MODEL), lambda i, *_: (i, 0)),
            pl.BlockSpec(memory_space=pl.ANY),
            pl.BlockSpec((1, 1, D_FF), lambda i, te, *_: (te[i], 0, 0)),
            pl.BlockSpec(memory_space=pl.ANY),
            pl.BlockSpec((1, 1, D_MODEL), lambda i, te, *_: (te[i], 0, 0)),
        ],
        out_specs=pl.BlockSpec((TILE, D_MODEL), lambda i, *_: (i, 0)),
        scratch_shapes=[
            pltpu.VMEM((2, D_FF, D_MODEL), jnp.float32),
            pltpu.VMEM((2, D_MODEL, D_FF), jnp.float32),
            pltpu.SemaphoreType.DMA((2,)),
            pltpu.SemaphoreType.DMA((2,)),
        ],
    )
    return pl.pallas_call(
        _ffn_body, grid_spec=grid_spec,
        out_shape=jax.ShapeDtypeStruct((NPAD, D_MODEL), jnp.float32),
    )(te, ta, xo, sl, npx, xs, W1, b1.reshape(E, 1, D_FF),
      W2, b2.reshape(E, 1, D_MODEL))


CH = 16
NCH = TPW // CH


def _sc_combine(x, ys, d1, d2, w1e, w2e):
    mesh = plsc.VectorSubcoreMesh(core_axis_name="c", subcore_axis_name="s")

    per_slot = [
        pltpu.VMEM((CH,), jnp.int32),
        pltpu.VMEM((CH,), jnp.int32),
        pltpu.VMEM((CH, 16), jnp.float32),
        pltpu.VMEM((CH, 16), jnp.float32),
        pltpu.VMEM((CH, D_MODEL), jnp.float32),
        pltpu.VMEM((CH, D_MODEL), jnp.float32),
        pltpu.VMEM((CH, D_MODEL), jnp.float32),
        pltpu.SemaphoreType.DMA,
        pltpu.SemaphoreType.DMA,
    ]

    @functools.partial(
        pl.kernel, mesh=mesh,
        out_type=jax.ShapeDtypeStruct((N, D_MODEL), jnp.float32),
        scratch_types=per_slot + per_slot,
    )
    def body(x_hbm, ys_hbm, d1_hbm, d2_hbm, w1_hbm, w2_hbm, out_hbm, *scr):
        slots = [scr[:9], scr[9:]]
        wid = lax.axis_index("s") * 2 + lax.axis_index("c")

        def issue(cnk, slot):
            idx1, idx2, w1r, w2r, xrows, buf1, buf2, semx, semg = slots[slot]
            base = wid * TPW + cnk * CH
            hx = pltpu.async_copy(x_hbm.at[pl.ds(base, CH)], xrows, semx)
            pltpu.sync_copy(d1_hbm.at[pl.ds(base, CH)], idx1)
            pltpu.sync_copy(d2_hbm.at[pl.ds(base, CH)], idx2)
            pltpu.sync_copy(w1_hbm.at[pl.ds(base, CH)], w1r)
            pltpu.sync_copy(w2_hbm.at[pl.ds(base, CH)], w2r)
            h1 = pltpu.async_copy(ys_hbm.at[idx1], buf1, semg)
            h2 = pltpu.async_copy(ys_hbm.at[idx2], buf2, semg)
            return (hx, h1, h2)

        hs = issue(0, 0)
        for cnk in range(NCH):
            slot = cnk % 2
            nhs = issue(cnk + 1, 1 - slot) if cnk + 1 < NCH else None
            for h in hs:
                h.wait()
            _, _, w1r, w2r, xrows, buf1, buf2, _, _ = slots[slot]

            @plsc.parallel_loop(0, CH, unroll=2)
            def _(t):
                ws1 = w1r[t, :]
                ws2 = w2r[t, :]
                for c in range(LANES):
                    sl = pl.ds(c * 16, 16)
                    xrows[t, sl] = (xrows[t, sl] + ws1 * buf1[t, sl]
                                    + ws2 * buf2[t, sl])
            base = wid * TPW + cnk * CH
            pltpu.sync_copy(xrows, out_hbm.at[pl.ds(base, CH)])
            hs = nhs

    return body(x, ys, d1, d2, w1e, w2e)


def kernel(x, gate_w, ln_g, ln_b, W1, b1, W2, b2):
    xn, d1, d2, w1e, w2e, te, ta, xo, sl, npx, aux = _router(x, gate_w, ln_g, ln_b)
    d1f = d1[:, 0]
    d2f = d2[:, 0]
    xs = _sc_scatter(xn, d1f, d2f)
    ys = _ffn(te[:, 0], ta[:, 0], xo[:, 0], sl[:, 0], npx[:, 0],
              xs, W1, b1, W2, b2)
    out = _sc_combine(x, ys, d1f, d2f, w1e, w2e)
    return out, aux[0, 0]
